# baseline (speedup 1.0000x reference)
import functools

import jax
import jax.numpy as jnp
from jax import lax
from jax.experimental import pallas as pl
from jax.experimental.pallas import tpu as pltpu
from jax.experimental.pallas import tpu_sc as plsc

HEADS = 4
K = 16
IN_DIM = 128
OUT_DIM = 128
HEAD_DIM = OUT_DIM // HEADS
B = 4
N = 2048

ROWS = 1024
NCH = 16
CHW = N // NCH


def _oe_merge_sort_pairs(n):
    pairs = []

    def merge(lo, n2, r):
        step = r * 2
        if step < n2:
            merge(lo, n2, step)
            merge(lo + r, n2, step)
            for i in range(lo + r, lo + n2 - r, step):
                pairs.append((i, i + r))
        else:
            pairs.append((lo, lo + r))

    def sort(lo, n2):
        if n2 > 1:
            m2 = n2 // 2
            sort(lo, m2)
            sort(lo + m2, m2)
            merge(lo, n2, 1)

    sort(0, n)
    return pairs
NUM_WORKERS = 32
CHUNK = N // NUM_WORKERS
GROUP = 8
NUM_GROUPS = CHUNK // GROUP


def _tc_body(pos_ref, posT_ref, x_ref, W_ref, attm_ref,
             idx_ref, h_ref, as_ref, an_ref):
    pos_b = pos_ref[...]
    posT = posT_ref[...]
    x_b = x_ref[...]
    W = W_ref[...]
    attm = attm_ref[...]

    sq_r = jnp.sum(pos_b * pos_b, axis=1, keepdims=True)
    sq_c = jnp.sum(posT * posT, axis=0, keepdims=True)
    dotp = jnp.dot(pos_b.astype(jnp.bfloat16), posT.astype(jnp.bfloat16),
                   preferred_element_type=jnp.float32)
    d2 = sq_r + sq_c - 2.0 * dotp

    cols = lax.broadcasted_iota(jnp.int32, d2.shape, 1)
    bits = lax.bitcast_convert_type(jnp.maximum(d2, 0.0), jnp.int32)
    key = jnp.bitwise_or(jnp.bitwise_and(bits, jnp.int32(-2048)), cols)

    ch = [key[:, s * CHW:(s + 1) * CHW] for s in range(NCH)]
    for i, j in _oe_merge_sort_pairs(NCH):
        lo = jnp.minimum(ch[i], ch[j])
        hi = jnp.maximum(ch[i], ch[j])
        ch[i], ch[j] = lo, hi
    lane = lax.broadcasted_iota(jnp.int32, (ROWS, CHW), 1)
    outs = []
    for t in range(K):
        m = jnp.min(ch[0], axis=1, keepdims=True)
        outs.append(jnp.bitwise_and(m, jnp.int32(2047)))
        if t < K - 1:
            lmask = lane == jnp.bitwise_and(m, jnp.int32(CHW - 1))
            for i in range(K - 1 - t):
                ch[i] = jnp.where(lmask, ch[i + 1], ch[i])
    idx_ref[...] = jnp.concatenate(outs, axis=1)

    h_b = jnp.dot(x_b.astype(jnp.bfloat16), W.astype(jnp.bfloat16),
                  preferred_element_type=jnp.float32)
    h_ref[...] = h_b
    ab = jnp.dot(h_b, attm, preferred_element_type=jnp.float32,
                 precision=lax.Precision.HIGHEST)
    as_ref[...] = ab[:, 0:4]
    an_ref[...] = ab[:, 4:8]


def _tc_stage(pos_pad, posT, x, W, attm):
    grid = (N // ROWS,)
    return pl.pallas_call(
        _tc_body,
        grid=grid,
        in_specs=[
            pl.BlockSpec((ROWS, 8), lambda r: (r, 0)),
            pl.BlockSpec((8, N), lambda r: (0, 0)),
            pl.BlockSpec((ROWS, IN_DIM), lambda r: (r, 0)),
            pl.BlockSpec((IN_DIM, OUT_DIM), lambda r: (0, 0)),
            pl.BlockSpec((OUT_DIM, 2 * HEADS), lambda r: (0, 0)),
        ],
        out_specs=[
            pl.BlockSpec((ROWS, K), lambda r: (r, 0)),
            pl.BlockSpec((ROWS, OUT_DIM), lambda r: (r, 0)),
            pl.BlockSpec((ROWS, HEADS), lambda r: (r, 0)),
            pl.BlockSpec((ROWS, HEADS), lambda r: (r, 0)),
        ],
        out_shape=[
            jax.ShapeDtypeStruct((N, K), jnp.int32),
            jax.ShapeDtypeStruct((N, OUT_DIM), jnp.float32),
            jax.ShapeDtypeStruct((N, HEADS), jnp.float32),
            jax.ShapeDtypeStruct((N, HEADS), jnp.float32),
        ],
    )(pos_pad, posT, x, W, attm)


def _sc_body(idx_hbm, asf_hbm, anf_hbm, x_hbm, h_hbm, out_hbm,
             idx_v, asf_v, an_v, x_v, out_v, rows_v, alpha_v, sem0, sem1):
    nc = 2
    wid = lax.axis_index("s") * nc + lax.axis_index("c")
    base = wid * CHUNK

    pltpu.sync_copy(idx_hbm.at[pl.ds(base * K, CHUNK * K)], idx_v)
    pltpu.sync_copy(asf_hbm.at[pl.ds(base * HEADS, CHUNK * HEADS)], asf_v)
    pltpu.sync_copy(anf_hbm, an_v)
    pltpu.sync_copy(x_hbm.at[pl.ds(base, CHUNK)], x_v)

    sems = (sem0, sem1)

    def start_gather(g, buf):
        dma = pltpu.make_async_copy(
            h_hbm.at[idx_v.at[pl.ds(g * GROUP * K, GROUP * K)]],
            rows_v.at[buf], sems[buf])
        dma.start()

    def wait_gather(buf):
        pltpu.make_async_copy(
            h_hbm.at[idx_v.at[pl.ds(0, GROUP * K)]],
            rows_v.at[buf], sems[buf]).wait()

    def compute_group(g, buf):
        def node_compute(i, _):
            node = g * GROUP + i
            nbr = idx_v[pl.ds(node * K, K)]
            an_idx = nbr * HEADS
            for h in range(HEADS):
                an_g = plsc.load_gather(an_v, [an_idx + h])
                as_b = plsc.load_gather(
                    asf_v, [jnp.zeros((K,), jnp.int32) + (node * HEADS + h)])
                s = as_b + an_g
                s = jnp.where(s > 0.0, s, 0.2 * s)
                e = jnp.exp(s - jnp.max(s))
                w = e / jnp.sum(e)
                alpha_v[...] = w

                def kstep(k, carry):
                    a0, a1 = carry
                    wk = plsc.load_gather(
                        alpha_v, [jnp.zeros((K,), jnp.int32) + k])
                    row = i * K + k
                    seg0 = rows_v[buf, row, pl.ds(h * HEAD_DIM, 16)]
                    seg1 = rows_v[buf, row, pl.ds(h * HEAD_DIM + 16, 16)]
                    return (a0 + wk * seg0, a1 + wk * seg1)

                acc0, acc1 = lax.fori_loop(
                    0, K, kstep,
                    (jnp.zeros((16,), jnp.float32),
                     jnp.zeros((16,), jnp.float32)), unroll=4)
                c0 = h * HEAD_DIM
                xa0 = x_v[node, pl.ds(c0, 16)]
                xa1 = x_v[node, pl.ds(c0 + 16, 16)]
                out_v[node, pl.ds(c0, 16)] = jnp.maximum(acc0 + xa0, 0.0)
                out_v[node, pl.ds(c0 + 16, 16)] = jnp.maximum(acc1 + xa1, 0.0)
            return ()

        lax.fori_loop(0, GROUP, node_compute, (), unroll=2)

    start_gather(0, 0)
    start_gather(1, 1)

    def pair(gg, _):
        g0 = 2 * gg
        wait_gather(0)
        compute_group(g0, 0)

        @pl.when(gg < NUM_GROUPS // 2 - 1)
        def _():
            start_gather(g0 + 2, 0)

        wait_gather(1)
        compute_group(g0 + 1, 1)

        @pl.when(gg < NUM_GROUPS // 2 - 1)
        def _():
            start_gather(g0 + 3, 1)
        return ()

    lax.fori_loop(0, NUM_GROUPS // 2, pair, (), unroll=False)

    pltpu.sync_copy(out_v, out_hbm.at[pl.ds(base, CHUNK)])


def _sc_stage(idx_flat, asf, anf, x2, h2):
    mesh = plsc.VectorSubcoreMesh(core_axis_name="c", subcore_axis_name="s")
    kern = functools.partial(
        pl.kernel,
        out_type=jax.ShapeDtypeStruct((N, OUT_DIM), jnp.float32),
        mesh=mesh,
        compiler_params=pltpu.CompilerParams(needs_layout_passes=False),
        scratch_types=[
            pltpu.VMEM((CHUNK * K,), jnp.int32),
            pltpu.VMEM((CHUNK * HEADS,), jnp.float32),
            pltpu.VMEM((N * HEADS,), jnp.float32),
            pltpu.VMEM((CHUNK, OUT_DIM), jnp.float32),
            pltpu.VMEM((CHUNK, OUT_DIM), jnp.float32),
            pltpu.VMEM((2, GROUP * K, OUT_DIM), jnp.float32),
            pltpu.VMEM((K,), jnp.float32),
            pltpu.SemaphoreType.DMA,
            pltpu.SemaphoreType.DMA,
        ],
    )(_sc_body)
    return kern(idx_flat, asf, anf, x2, h2)


def kernel(x, pos, W, att):
    pos_pad = jnp.concatenate(
        [pos, jnp.zeros((B, N, 5), jnp.float32)], axis=-1)
    posT = jnp.swapaxes(pos_pad, 1, 2)

    att_l = att[0, :, :HEAD_DIM]
    att_r = att[0, :, HEAD_DIM:]
    eye = jnp.eye(HEADS, dtype=jnp.float32)
    attm_l = (att_l[:, :, None] * eye[:, None, :]).reshape(OUT_DIM, HEADS)
    attm_r = (att_r[:, :, None] * eye[:, None, :]).reshape(OUT_DIM, HEADS)
    attm = jnp.concatenate([attm_l, attm_r], axis=1)

    outs = []
    for b in range(B):
        idx, h, a_self, a_nbr = _tc_stage(pos_pad[b], posT[b], x[b], W, attm)
        out_b = _sc_stage(idx.reshape(-1), a_self.reshape(-1),
                          a_nbr.reshape(-1), x[b], h)
        outs.append(out_b)
    return jnp.stack(outs, axis=0)


# speedup vs baseline: 1.0012x; 1.0012x over previous
import functools

import jax
import jax.numpy as jnp
from jax import lax
from jax.experimental import pallas as pl
from jax.experimental.pallas import tpu as pltpu
from jax.experimental.pallas import tpu_sc as plsc

HEADS = 4
K = 16
IN_DIM = 128
OUT_DIM = 128
HEAD_DIM = OUT_DIM // HEADS
B = 4
N = 2048

ROWS = 1024
NCH = 16
CHW = N // NCH


def _oe_merge_sort_pairs(n):
    pairs = []

    def merge(lo, n2, r):
        step = r * 2
        if step < n2:
            merge(lo, n2, step)
            merge(lo + r, n2, step)
            for i in range(lo + r, lo + n2 - r, step):
                pairs.append((i, i + r))
        else:
            pairs.append((lo, lo + r))

    def sort(lo, n2):
        if n2 > 1:
            m2 = n2 // 2
            sort(lo, m2)
            sort(lo + m2, m2)
            merge(lo, n2, 1)

    sort(0, n)
    return pairs
NUM_WORKERS = 32
CHUNK = N // NUM_WORKERS
GROUP = 8
NUM_GROUPS = CHUNK // GROUP


def _tc_body(pos_ref, posT_ref, x_ref, W_ref, attm_ref,
             idx_ref, h_ref, as_ref, an_ref):
    pos_b = pos_ref[...]
    posT = posT_ref[...]
    x_b = x_ref[...]
    W = W_ref[...]
    attm = attm_ref[...]

    sq_r = jnp.sum(pos_b * pos_b, axis=1, keepdims=True)
    sq_c = jnp.sum(posT * posT, axis=0, keepdims=True)
    dotp = jnp.dot(pos_b.astype(jnp.bfloat16), posT.astype(jnp.bfloat16),
                   preferred_element_type=jnp.float32)
    d2 = sq_r + sq_c - 2.0 * dotp

    cols = lax.broadcasted_iota(jnp.int32, d2.shape, 1)
    bits = lax.bitcast_convert_type(jnp.maximum(d2, 0.0), jnp.int32)
    key = jnp.bitwise_or(jnp.bitwise_and(bits, jnp.int32(-2048)), cols)

    ch = [key[:, s * CHW:(s + 1) * CHW] for s in range(NCH)]
    for i, j in _oe_merge_sort_pairs(NCH):
        lo = jnp.minimum(ch[i], ch[j])
        hi = jnp.maximum(ch[i], ch[j])
        ch[i], ch[j] = lo, hi
    lane = lax.broadcasted_iota(jnp.int32, (ROWS, CHW), 1)
    outs = []
    for t in range(K):
        m = jnp.min(ch[0], axis=1, keepdims=True)
        outs.append(jnp.bitwise_and(m, jnp.int32(2047)))
        if t < K - 1:
            lmask = lane == jnp.bitwise_and(m, jnp.int32(CHW - 1))
            for i in range(K - 1 - t):
                ch[i] = jnp.where(lmask, ch[i + 1], ch[i])
    idx_ref[...] = jnp.concatenate(outs, axis=1)

    h_b = jnp.dot(x_b.astype(jnp.bfloat16), W.astype(jnp.bfloat16),
                  preferred_element_type=jnp.float32)
    h_ref[...] = h_b
    ab = jnp.dot(h_b, attm, preferred_element_type=jnp.float32,
                 precision=lax.Precision.HIGHEST)
    as_ref[...] = ab[:, 0:4]
    an_ref[...] = ab[:, 4:8]


def _tc_stage(pos_pad, posT, x, W, attm):
    grid = (N // ROWS,)
    return pl.pallas_call(
        _tc_body,
        grid=grid,
        in_specs=[
            pl.BlockSpec((ROWS, 8), lambda r: (r, 0)),
            pl.BlockSpec((8, N), lambda r: (0, 0)),
            pl.BlockSpec((ROWS, IN_DIM), lambda r: (r, 0)),
            pl.BlockSpec((IN_DIM, OUT_DIM), lambda r: (0, 0)),
            pl.BlockSpec((OUT_DIM, 2 * HEADS), lambda r: (0, 0)),
        ],
        out_specs=[
            pl.BlockSpec((ROWS, K), lambda r: (r, 0)),
            pl.BlockSpec((ROWS, OUT_DIM), lambda r: (r, 0)),
            pl.BlockSpec((ROWS, HEADS), lambda r: (r, 0)),
            pl.BlockSpec((ROWS, HEADS), lambda r: (r, 0)),
        ],
        out_shape=[
            jax.ShapeDtypeStruct((N, K), jnp.int32),
            jax.ShapeDtypeStruct((N, OUT_DIM), jnp.float32),
            jax.ShapeDtypeStruct((N, HEADS), jnp.float32),
            jax.ShapeDtypeStruct((N, HEADS), jnp.float32),
        ],
    )(pos_pad, posT, x, W, attm)


def _sc_body(idx_hbm, asf_hbm, anf_hbm, x_hbm, h_hbm, out_hbm,
             idx_v, asf_v, an_v, x_v, out_v, rows_v, alpha_v, sem0, sem1):
    nc = 2
    wid = lax.axis_index("s") * nc + lax.axis_index("c")
    base = wid * CHUNK

    pltpu.sync_copy(idx_hbm.at[pl.ds(base * K, CHUNK * K)], idx_v)
    pltpu.sync_copy(asf_hbm.at[pl.ds(base * HEADS, CHUNK * HEADS)], asf_v)
    pltpu.sync_copy(anf_hbm, an_v)
    pltpu.sync_copy(x_hbm.at[pl.ds(base, CHUNK)], x_v)

    sems = (sem0, sem1)

    def start_gather(g, buf):
        dma = pltpu.make_async_copy(
            h_hbm.at[idx_v.at[pl.ds(g * GROUP * K, GROUP * K)]],
            rows_v.at[buf], sems[buf])
        dma.start()

    def wait_gather(buf):
        pltpu.make_async_copy(
            h_hbm.at[idx_v.at[pl.ds(0, GROUP * K)]],
            rows_v.at[buf], sems[buf]).wait()

    def compute_group(g, buf):
        def node_compute(i, _):
            node = g * GROUP + i
            nbr = idx_v[pl.ds(node * K, K)]
            an_idx = nbr * HEADS
            for h in range(HEADS):
                an_g = plsc.load_gather(an_v, [an_idx + h])
                as_b = plsc.load_gather(
                    asf_v, [jnp.zeros((K,), jnp.int32) + (node * HEADS + h)])
                s = as_b + an_g
                s = jnp.where(s > 0.0, s, 0.2 * s)
                e = jnp.exp(s - jnp.max(s))
                w = e / jnp.sum(e)
                alpha_v[...] = w

                def kstep(k, carry):
                    a0, a1 = carry
                    wk = plsc.load_gather(
                        alpha_v, [jnp.zeros((K,), jnp.int32) + k])
                    row = i * K + k
                    seg0 = rows_v[buf, row, pl.ds(h * HEAD_DIM, 16)]
                    seg1 = rows_v[buf, row, pl.ds(h * HEAD_DIM + 16, 16)]
                    return (a0 + wk * seg0, a1 + wk * seg1)

                acc0, acc1 = lax.fori_loop(
                    0, K, kstep,
                    (jnp.zeros((16,), jnp.float32),
                     jnp.zeros((16,), jnp.float32)), unroll=4)
                c0 = h * HEAD_DIM
                xa0 = x_v[node, pl.ds(c0, 16)]
                xa1 = x_v[node, pl.ds(c0 + 16, 16)]
                out_v[node, pl.ds(c0, 16)] = jnp.maximum(acc0 + xa0, 0.0)
                out_v[node, pl.ds(c0 + 16, 16)] = jnp.maximum(acc1 + xa1, 0.0)
            return ()

        lax.fori_loop(0, GROUP, node_compute, (), unroll=False)

    start_gather(0, 0)
    start_gather(1, 1)

    def pair(gg, _):
        g0 = 2 * gg
        wait_gather(0)
        compute_group(g0, 0)

        @pl.when(gg < NUM_GROUPS // 2 - 1)
        def _():
            start_gather(g0 + 2, 0)

        wait_gather(1)
        compute_group(g0 + 1, 1)

        @pl.when(gg < NUM_GROUPS // 2 - 1)
        def _():
            start_gather(g0 + 3, 1)
        return ()

    lax.fori_loop(0, NUM_GROUPS // 2, pair, (), unroll=False)

    pltpu.sync_copy(out_v, out_hbm.at[pl.ds(base, CHUNK)])


def _sc_stage(idx_flat, asf, anf, x2, h2):
    mesh = plsc.VectorSubcoreMesh(core_axis_name="c", subcore_axis_name="s")
    kern = functools.partial(
        pl.kernel,
        out_type=jax.ShapeDtypeStruct((N, OUT_DIM), jnp.float32),
        mesh=mesh,
        compiler_params=pltpu.CompilerParams(needs_layout_passes=False),
        scratch_types=[
            pltpu.VMEM((CHUNK * K,), jnp.int32),
            pltpu.VMEM((CHUNK * HEADS,), jnp.float32),
            pltpu.VMEM((N * HEADS,), jnp.float32),
            pltpu.VMEM((CHUNK, OUT_DIM), jnp.float32),
            pltpu.VMEM((CHUNK, OUT_DIM), jnp.float32),
            pltpu.VMEM((2, GROUP * K, OUT_DIM), jnp.float32),
            pltpu.VMEM((K,), jnp.float32),
            pltpu.SemaphoreType.DMA,
            pltpu.SemaphoreType.DMA,
        ],
    )(_sc_body)
    return kern(idx_flat, asf, anf, x2, h2)


def kernel(x, pos, W, att):
    pos_pad = jnp.concatenate(
        [pos, jnp.zeros((B, N, 5), jnp.float32)], axis=-1)
    posT = jnp.swapaxes(pos_pad, 1, 2)

    att_l = att[0, :, :HEAD_DIM]
    att_r = att[0, :, HEAD_DIM:]
    eye = jnp.eye(HEADS, dtype=jnp.float32)
    attm_l = (att_l[:, :, None] * eye[:, None, :]).reshape(OUT_DIM, HEADS)
    attm_r = (att_r[:, :, None] * eye[:, None, :]).reshape(OUT_DIM, HEADS)
    attm = jnp.concatenate([attm_l, attm_r], axis=1)

    outs = []
    for b in range(B):
        idx, h, a_self, a_nbr = _tc_stage(pos_pad[b], posT[b], x[b], W, attm)
        out_b = _sc_stage(idx.reshape(-1), a_self.reshape(-1),
                          a_nbr.reshape(-1), x[b], h)
        outs.append(out_b)
    return jnp.stack(outs, axis=0)


# speedup vs baseline: 1.0096x; 1.0084x over previous
import functools

import jax
import jax.numpy as jnp
from jax import lax
from jax.experimental import pallas as pl
from jax.experimental.pallas import tpu as pltpu
from jax.experimental.pallas import tpu_sc as plsc

HEADS = 4
K = 16
IN_DIM = 128
OUT_DIM = 128
HEAD_DIM = OUT_DIM // HEADS
B = 4
N = 2048

ROWS = 1024
NCH = 16
CHW = N // NCH


def _oe_merge_sort_pairs(n):
    pairs = []

    def merge(lo, n2, r):
        step = r * 2
        if step < n2:
            merge(lo, n2, step)
            merge(lo + r, n2, step)
            for i in range(lo + r, lo + n2 - r, step):
                pairs.append((i, i + r))
        else:
            pairs.append((lo, lo + r))

    def sort(lo, n2):
        if n2 > 1:
            m2 = n2 // 2
            sort(lo, m2)
            sort(lo + m2, m2)
            merge(lo, n2, 1)

    sort(0, n)
    return pairs
NUM_WORKERS = 32
CHUNK = N // NUM_WORKERS
GROUP = 8
NUM_GROUPS = CHUNK // GROUP


def _tc_body(pos_ref, posT_ref, x_ref, W_ref, attm_ref,
             idx_ref, h_ref, as_ref, an_ref):
    pos_b = pos_ref[...]
    posT = posT_ref[...]
    x_b = x_ref[...]
    W = W_ref[...]
    attm = attm_ref[...]

    sq_r = jnp.sum(pos_b * pos_b, axis=1, keepdims=True)
    sq_c = jnp.sum(posT * posT, axis=0, keepdims=True)
    dotp = jnp.dot(pos_b.astype(jnp.bfloat16), posT.astype(jnp.bfloat16),
                   preferred_element_type=jnp.float32)
    d2 = sq_r + sq_c - 2.0 * dotp

    cols = lax.broadcasted_iota(jnp.int32, d2.shape, 1)
    bits = lax.bitcast_convert_type(jnp.maximum(d2, 0.0), jnp.int32)
    key = jnp.bitwise_or(jnp.bitwise_and(bits, jnp.int32(-2048)), cols)

    ch = [key[:, s * CHW:(s + 1) * CHW] for s in range(NCH)]
    for i, j in _oe_merge_sort_pairs(NCH):
        lo = jnp.minimum(ch[i], ch[j])
        hi = jnp.maximum(ch[i], ch[j])
        ch[i], ch[j] = lo, hi
    lane = lax.broadcasted_iota(jnp.int32, (ROWS, CHW), 1)
    outs = []
    for t in range(K):
        m = jnp.min(ch[0], axis=1, keepdims=True)
        outs.append(jnp.bitwise_and(m, jnp.int32(2047)))
        if t < K - 1:
            lmask = lane == jnp.bitwise_and(m, jnp.int32(CHW - 1))
            for i in range(K - 1 - t):
                ch[i] = jnp.where(lmask, ch[i + 1], ch[i])
    idx_ref[...] = jnp.concatenate(outs, axis=1)

    h_b = jnp.dot(x_b.astype(jnp.bfloat16), W.astype(jnp.bfloat16),
                  preferred_element_type=jnp.float32)
    h_ref[...] = h_b
    ab = jnp.dot(h_b, attm, preferred_element_type=jnp.float32,
                 precision=lax.Precision.HIGHEST)
    as_ref[...] = ab[:, 0:4]
    an_ref[...] = ab[:, 4:8]


def _tc_stage(pos_pad, posT, x, W, attm):
    grid = (N // ROWS,)
    return pl.pallas_call(
        _tc_body,
        grid=grid,
        in_specs=[
            pl.BlockSpec((ROWS, 8), lambda r: (r, 0)),
            pl.BlockSpec((8, N), lambda r: (0, 0)),
            pl.BlockSpec((ROWS, IN_DIM), lambda r: (r, 0)),
            pl.BlockSpec((IN_DIM, OUT_DIM), lambda r: (0, 0)),
            pl.BlockSpec((OUT_DIM, 2 * HEADS), lambda r: (0, 0)),
        ],
        out_specs=[
            pl.BlockSpec((ROWS, K), lambda r: (r, 0)),
            pl.BlockSpec((ROWS, OUT_DIM), lambda r: (r, 0)),
            pl.BlockSpec((ROWS, HEADS), lambda r: (r, 0)),
            pl.BlockSpec((ROWS, HEADS), lambda r: (r, 0)),
        ],
        out_shape=[
            jax.ShapeDtypeStruct((N, K), jnp.int32),
            jax.ShapeDtypeStruct((N, OUT_DIM), jnp.float32),
            jax.ShapeDtypeStruct((N, HEADS), jnp.float32),
            jax.ShapeDtypeStruct((N, HEADS), jnp.float32),
        ],
    )(pos_pad, posT, x, W, attm)


def _sc_body(idx_hbm, asf_hbm, anf_hbm, x_hbm, h_hbm, out_hbm,
             idx_v, asf_v, an_v, x_v, out_v, rows_v, alpha_v,
             sem0, sem1, sem2):
    nc = 2
    wid = lax.axis_index("s") * nc + lax.axis_index("c")
    base = wid * CHUNK

    pltpu.sync_copy(idx_hbm.at[pl.ds(base * K, CHUNK * K)], idx_v)

    sems = (sem0, sem1)

    def start_gather(g, buf):
        dma = pltpu.make_async_copy(
            h_hbm.at[idx_v.at[pl.ds(g * GROUP * K, GROUP * K)]],
            rows_v.at[buf], sems[buf])
        dma.start()

    def wait_gather(buf):
        pltpu.make_async_copy(
            h_hbm.at[idx_v.at[pl.ds(0, GROUP * K)]],
            rows_v.at[buf], sems[buf]).wait()

    def compute_group(g, buf):
        def node_compute(i, _):
            node = g * GROUP + i
            nbr = idx_v[pl.ds(node * K, K)]
            an_idx = nbr * HEADS
            for h in range(HEADS):
                an_g = plsc.load_gather(an_v, [an_idx + h])
                as_b = plsc.load_gather(
                    asf_v, [jnp.zeros((K,), jnp.int32) + (node * HEADS + h)])
                s = as_b + an_g
                s = jnp.where(s > 0.0, s, 0.2 * s)
                e = jnp.exp(s - jnp.max(s))
                w = e / jnp.sum(e)
                alpha_v[...] = w

                def kstep(k, carry):
                    a0, a1 = carry
                    wk = plsc.load_gather(
                        alpha_v, [jnp.zeros((K,), jnp.int32) + k])
                    row = i * K + k
                    seg0 = rows_v[buf, row, pl.ds(h * HEAD_DIM, 16)]
                    seg1 = rows_v[buf, row, pl.ds(h * HEAD_DIM + 16, 16)]
                    return (a0 + wk * seg0, a1 + wk * seg1)

                acc0, acc1 = lax.fori_loop(
                    0, K, kstep,
                    (jnp.zeros((16,), jnp.float32),
                     jnp.zeros((16,), jnp.float32)), unroll=4)
                c0 = h * HEAD_DIM
                xa0 = x_v[node, pl.ds(c0, 16)]
                xa1 = x_v[node, pl.ds(c0 + 16, 16)]
                out_v[node, pl.ds(c0, 16)] = jnp.maximum(acc0 + xa0, 0.0)
                out_v[node, pl.ds(c0 + 16, 16)] = jnp.maximum(acc1 + xa1, 0.0)
            return ()

        lax.fori_loop(0, GROUP, node_compute, (), unroll=False)

    start_gather(0, 0)
    start_gather(1, 1)

    stage = [
        pltpu.make_async_copy(
            asf_hbm.at[pl.ds(base * HEADS, CHUNK * HEADS)], asf_v, sem2),
        pltpu.make_async_copy(anf_hbm, an_v, sem2),
        pltpu.make_async_copy(x_hbm.at[pl.ds(base, CHUNK)], x_v, sem2),
    ]
    for s_ in stage:
        s_.start()
    for s_ in stage:
        s_.wait()

    def pair(gg, _):
        g0 = 2 * gg
        wait_gather(0)
        compute_group(g0, 0)

        @pl.when(gg < NUM_GROUPS // 2 - 1)
        def _():
            start_gather(g0 + 2, 0)

        wait_gather(1)
        compute_group(g0 + 1, 1)

        @pl.when(gg < NUM_GROUPS // 2 - 1)
        def _():
            start_gather(g0 + 3, 1)
        return ()

    lax.fori_loop(0, NUM_GROUPS // 2, pair, (), unroll=False)

    pltpu.sync_copy(out_v, out_hbm.at[pl.ds(base, CHUNK)])


def _sc_stage(idx_flat, asf, anf, x2, h2):
    mesh = plsc.VectorSubcoreMesh(core_axis_name="c", subcore_axis_name="s")
    kern = functools.partial(
        pl.kernel,
        out_type=jax.ShapeDtypeStruct((N, OUT_DIM), jnp.float32),
        mesh=mesh,
        compiler_params=pltpu.CompilerParams(needs_layout_passes=False),
        scratch_types=[
            pltpu.VMEM((CHUNK * K,), jnp.int32),
            pltpu.VMEM((CHUNK * HEADS,), jnp.float32),
            pltpu.VMEM((N * HEADS,), jnp.float32),
            pltpu.VMEM((CHUNK, OUT_DIM), jnp.float32),
            pltpu.VMEM((CHUNK, OUT_DIM), jnp.float32),
            pltpu.VMEM((2, GROUP * K, OUT_DIM), jnp.float32),
            pltpu.VMEM((K,), jnp.float32),
            pltpu.SemaphoreType.DMA,
            pltpu.SemaphoreType.DMA,
            pltpu.SemaphoreType.DMA,
        ],
    )(_sc_body)
    return kern(idx_flat, asf, anf, x2, h2)


def kernel(x, pos, W, att):
    pos_pad = jnp.concatenate(
        [pos, jnp.zeros((B, N, 5), jnp.float32)], axis=-1)
    posT = jnp.swapaxes(pos_pad, 1, 2)

    att_l = att[0, :, :HEAD_DIM]
    att_r = att[0, :, HEAD_DIM:]
    eye = jnp.eye(HEADS, dtype=jnp.float32)
    attm_l = (att_l[:, :, None] * eye[:, None, :]).reshape(OUT_DIM, HEADS)
    attm_r = (att_r[:, :, None] * eye[:, None, :]).reshape(OUT_DIM, HEADS)
    attm = jnp.concatenate([attm_l, attm_r], axis=1)

    outs = []
    for b in range(B):
        idx, h, a_self, a_nbr = _tc_stage(pos_pad[b], posT[b], x[b], W, attm)
        out_b = _sc_stage(idx.reshape(-1), a_self.reshape(-1),
                          a_nbr.reshape(-1), x[b], h)
        outs.append(out_b)
    return jnp.stack(outs, axis=0)
